# baseline (speedup 1.0000x reference)
import functools

import jax
import jax.numpy as jnp
from jax.experimental import pallas as pl


_BLKR = 4096


def _body(xs_ref, xt_ref, w1_ref, b1_ref, w2_ref, b2_ref, out_ref, *, nb_per_batch, B):
    i = pl.program_id(0)

    @pl.when(i == 0)
    def _init():
        out_ref[...] = jnp.zeros_like(out_ref)

    w1 = w1_ref[...]
    b1 = b1_ref[...]
    w2 = w2_ref[...]
    b2 = b2_ref[...]

    def agg(x):
        h = jnp.maximum(jnp.dot(x, w1, preferred_element_type=jnp.float32) + b1, 0.0)
        o = jnp.maximum(jnp.dot(h, w2, preferred_element_type=jnp.float32) + b2, 0.0)
        return jnp.sum(o)

    part = agg(xs_ref[...]) - agg(xt_ref[...])
    b = i // nb_per_batch
    onehot = jax.lax.broadcasted_iota(jnp.int32, (1, B), 1) == b
    out_ref[...] += jnp.where(onehot, part, 0.0)


def kernel(student_diagrams, teacher_diagrams, W1, b1, W2, b2):
    B, N, D = student_diagrams.shape
    H = W1.shape[1]
    xs = student_diagrams.reshape(B * N, D)
    xt = teacher_diagrams.reshape(B * N, D)
    nb_per_batch = N // _BLKR
    grid = (B * N // _BLKR,)
    out = pl.pallas_call(
        functools.partial(_body, nb_per_batch=nb_per_batch, B=B),
        grid=grid,
        in_specs=[
            pl.BlockSpec((_BLKR, D), lambda i: (i, 0)),
            pl.BlockSpec((_BLKR, D), lambda i: (i, 0)),
            pl.BlockSpec((D, H), lambda i: (0, 0)),
            pl.BlockSpec((1, H), lambda i: (0, 0)),
            pl.BlockSpec((H, 1), lambda i: (0, 0)),
            pl.BlockSpec((1, 1), lambda i: (0, 0)),
        ],
        out_specs=pl.BlockSpec((1, B), lambda i: (0, 0)),
        out_shape=jax.ShapeDtypeStruct((1, B), jnp.float32),
    )(xs, xt, W1, b1.reshape(1, H), W2, b2.reshape(1, 1))
    diff = out[0]
    return jnp.mean(diff * diff)


# speedup vs baseline: 33.8126x; 33.8126x over previous
import functools

import jax
import jax.numpy as jnp
from jax.experimental import pallas as pl


_BLKN = 8192


def _body(xs_ref, xt_ref, w1a_ref, w2_ref, b2_ref, out_ref, *, B):
    b = pl.program_id(0)
    j = pl.program_id(1)

    @pl.when(jnp.logical_and(b == 0, j == 0))
    def _init():
        out_ref[...] = jnp.zeros_like(out_ref)

    w1a = w1a_ref[...]
    w2 = w2_ref[...]
    b2 = b2_ref[...]
    ones = jnp.ones((1, _BLKN), jnp.float32)

    def agg(x3):
        x4 = jnp.concatenate([x3, ones], axis=0)
        h = jax.lax.dot_general(w1a, x4, (((1,), (0,)), ((), ())),
                                preferred_element_type=jnp.float32)
        h = jnp.maximum(h, 0.0)
        o = jax.lax.dot_general(w2, h, (((1,), (0,)), ((), ())),
                                preferred_element_type=jnp.float32)
        o = jnp.maximum(o + b2, 0.0)
        return jnp.sum(o)

    part = agg(xs_ref[0]) - agg(xt_ref[0])
    onehot = jax.lax.broadcasted_iota(jnp.int32, (1, B), 1) == b
    out_ref[...] += jnp.where(onehot, part, 0.0)


def kernel(student_diagrams, teacher_diagrams, W1, b1, W2, b2):
    B, N, D = student_diagrams.shape
    H = W1.shape[1]
    xs = jnp.swapaxes(student_diagrams, 1, 2)
    xt = jnp.swapaxes(teacher_diagrams, 1, 2)
    w1a = jnp.concatenate([W1.T, b1[:, None]], axis=1)
    grid = (B, N // _BLKN)
    out = pl.pallas_call(
        functools.partial(_body, B=B),
        grid=grid,
        in_specs=[
            pl.BlockSpec((1, D, _BLKN), lambda b, j: (b, 0, j)),
            pl.BlockSpec((1, D, _BLKN), lambda b, j: (b, 0, j)),
            pl.BlockSpec((H, D + 1), lambda b, j: (0, 0)),
            pl.BlockSpec((1, H), lambda b, j: (0, 0)),
            pl.BlockSpec((1, 1), lambda b, j: (0, 0)),
        ],
        out_specs=pl.BlockSpec((1, B), lambda b, j: (0, 0)),
        out_shape=jax.ShapeDtypeStruct((1, B), jnp.float32),
    )(xs, xt, w1a, W2.T, b2.reshape(1, 1))
    diff = out[0]
    return jnp.mean(diff * diff)
